# Initial kernel scaffold; baseline (speedup 1.0000x reference)
import jax
import jax.numpy as jnp
from jax.experimental import pallas as pl


def kernel(z, sldj, labels, means):
    raise NotImplementedError("write your pallas kernel here")


# speedup vs baseline: 3.0532x; 3.0532x over previous
import functools

import jax
import jax.numpy as jnp
import numpy as np
from jax import lax
from jax.experimental import pallas as pl

NUM_ATTR = 8
NUM_CLASSES = 1000
TOTAL_DIM = 3072
DIMS_PER_ATTR = TOTAL_DIM // NUM_ATTR
BATCH = 4096
BB = 512


def _body(z_ref, sldj_ref, lab_ref, means_ref, out_ref):
    pid = pl.program_id(0)

    @pl.when(pid == 0)
    def _init():
        out_ref[...] = jnp.reshape(-jnp.sum(sldj_ref[...]), (1, 1))

    z = z_ref[...]
    acc = 0.5 * jnp.sum(z * z)
    labs = lab_ref[...]
    class_iota = lax.broadcasted_iota(jnp.int32, (BB, NUM_CLASSES), 1)
    for i in range(NUM_ATTR):
        onehot = (labs[:, i][:, None] == class_iota).astype(jnp.bfloat16)
        sel = jnp.dot(onehot, means_ref[i],
                      preferred_element_type=jnp.float32)
        zseg = z[:, i * DIMS_PER_ATTR:(i + 1) * DIMS_PER_ATTR]
        acc += 0.5 * jnp.sum(sel * sel) - jnp.sum(zseg * sel)
    out_ref[...] += jnp.reshape(acc, (1, 1))


@jax.jit
def kernel(z, sldj, labels, means):
    means_bf = means.astype(jnp.bfloat16)
    labels = labels.astype(jnp.int32)
    sldj2d = sldj.reshape(32, BATCH // 32)
    grid = BATCH // BB
    out = pl.pallas_call(
        _body,
        grid=(grid,),
        in_specs=[
            pl.BlockSpec((BB, TOTAL_DIM), lambda b: (b, 0)),
            pl.BlockSpec((32, BATCH // 32), lambda b: (0, 0)),
            pl.BlockSpec((BB, NUM_ATTR), lambda b: (b, 0)),
            pl.BlockSpec((NUM_ATTR, NUM_CLASSES, DIMS_PER_ATTR),
                         lambda b: (0, 0, 0)),
        ],
        out_specs=pl.BlockSpec((1, 1), lambda b: (0, 0)),
        out_shape=jax.ShapeDtypeStruct((1, 1), jnp.float32),
    )(z, sldj2d, labels, means_bf)
    const = 0.5 * TOTAL_DIM * np.log(2 * np.pi)
    return out[0, 0] / BATCH + jnp.float32(const)
